# baseline (device time: 102756 ns/iter reference)
import jax
import jax.numpy as jnp
from jax import lax
from jax.experimental import pallas as pl
from jax.experimental.pallas import tpu as pltpu

H = 16
SHARE = H // 2
D = 128
S_LOCAL = 1024
SCALE = D ** -0.5


def kernel(Q, K, V):
    my_x = lax.axis_index("x")

    def share(A):
        return lax.dynamic_slice_in_dim(
            A[0], my_x * SHARE, SHARE, axis=1
        ).astype(jnp.bfloat16)

    q = share(Q)
    k = jnp.transpose(share(K), (1, 0, 2))
    v = jnp.transpose(share(V), (1, 0, 2))

    def body(q_ref, k_ref, v_ref, o_ref, ko_ref, vo_ref, obuf_ref, oin_ref,
             k_send, k_recv, v_send, v_recv, o_send, o_recv):
        x_idx = lax.axis_index("x")
        my_y = lax.axis_index("y")

        barrier = pltpu.get_barrier_semaphore()
        for nbr in ((x_idx, 1 - my_y), (1 - x_idx, my_y)):
            pl.semaphore_signal(
                barrier, inc=1, device_id=nbr,
                device_id_type=pl.DeviceIdType.MESH,
            )
        pl.semaphore_wait(barrier, 2)

        def run_column(x):
            y_peer = (x, 1 - my_y)
            x_peer = (1 - x, my_y)
            mine_off = x * SHARE
            twin_off = (1 - x) * SHARE

            def rdma(src, dst, send, recv, i, peer):
                return pltpu.make_async_remote_copy(
                    src_ref=src.at[i], dst_ref=dst.at[i],
                    send_sem=send.at[i], recv_sem=recv.at[i],
                    device_id=peer, device_id_type=pl.DeviceIdType.MESH,
                )

            kx = [
                rdma(k_ref, ko_ref, k_send, k_recv, i, y_peer)
                for i in range(SHARE)
            ]
            vx = [
                rdma(v_ref, vo_ref, v_send, v_recv, i, y_peer)
                for i in range(SHARE)
            ]
            for i in range(SHARE):
                kx[i].start()
                vx[i].start()

            o_ship = []
            for i in range(SHARE):
                qh = q_ref[:, i, :]
                s0 = lax.dot_general(
                    qh, k_ref[i], (((1,), (1,)), ((), ())),
                    preferred_element_type=jnp.float32,
                )
                p0 = jnp.exp(s0 * SCALE)
                l0 = jnp.sum(p0, axis=1, keepdims=True)
                o0 = lax.dot_general(
                    p0.astype(jnp.bfloat16), v_ref[i],
                    (((1,), (0,)), ((), ())),
                    preferred_element_type=jnp.float32,
                )
                kx[i].wait()
                s1 = lax.dot_general(
                    qh, ko_ref[i], (((1,), (1,)), ((), ())),
                    preferred_element_type=jnp.float32,
                )
                p1 = jnp.exp(s1 * SCALE)
                l1 = jnp.sum(p1, axis=1, keepdims=True)
                vx[i].wait()
                o1 = lax.dot_general(
                    p1.astype(jnp.bfloat16), vo_ref[i],
                    (((1,), (0,)), ((), ())),
                    preferred_element_type=jnp.float32,
                )
                o_head = (o0 + o1) / (l0 + l1)
                o_ref[:, mine_off + i, :] = o_head
                obuf_ref[i] = o_head.astype(jnp.bfloat16)
                ship = rdma(obuf_ref, oin_ref, o_send, o_recv, i, x_peer)
                ship.start()
                o_ship.append(ship)

            for i in range(SHARE):
                rdma(obuf_ref, oin_ref, o_send, o_recv, i, x_peer).wait_recv()
                o_ref[:, twin_off + i, :] = oin_ref[i].astype(jnp.float32)

            for ship in o_ship:
                ship.wait_send()

        @pl.when(x_idx == 0)
        def _():
            run_column(0)

        @pl.when(x_idx == 1)
        def _():
            run_column(1)

    o = pl.pallas_call(
        body,
        out_shape=jax.ShapeDtypeStruct((S_LOCAL, H, D), jnp.float32),
        in_specs=[pl.BlockSpec(memory_space=pltpu.VMEM)] * 3,
        out_specs=pl.BlockSpec(memory_space=pltpu.VMEM),
        scratch_shapes=[
            pltpu.VMEM((SHARE, S_LOCAL, D), jnp.bfloat16),
            pltpu.VMEM((SHARE, S_LOCAL, D), jnp.bfloat16),
            pltpu.VMEM((SHARE, S_LOCAL, D), jnp.bfloat16),
            pltpu.VMEM((SHARE, S_LOCAL, D), jnp.bfloat16),
            pltpu.SemaphoreType.DMA((SHARE,)),
            pltpu.SemaphoreType.DMA((SHARE,)),
            pltpu.SemaphoreType.DMA((SHARE,)),
            pltpu.SemaphoreType.DMA((SHARE,)),
            pltpu.SemaphoreType.DMA((SHARE,)),
            pltpu.SemaphoreType.DMA((SHARE,)),
        ],
        compiler_params=pltpu.CompilerParams(
            collective_id=0, vmem_limit_bytes=60 * 1024 * 1024
        ),
    )(q, k, v)

    return o[None]


# device time: 101828 ns/iter; 1.0091x vs baseline; 1.0091x over previous
import jax
import jax.numpy as jnp
from jax import lax
from jax.experimental import pallas as pl
from jax.experimental.pallas import tpu as pltpu

H = 16
SHARE = H // 2
D = 128
S_LOCAL = 1024
SCALE = D ** -0.5


def kernel(Q, K, V):
    my_x = lax.axis_index("x")

    def share(A):
        return lax.dynamic_slice_in_dim(A[0], my_x * SHARE, SHARE, axis=1)

    q = (share(Q) * SCALE).astype(jnp.bfloat16)
    k = jnp.transpose(share(K).astype(jnp.bfloat16), (1, 0, 2))
    v = jnp.transpose(share(V).astype(jnp.bfloat16), (1, 0, 2))
    v8 = v.astype(jnp.float8_e4m3fn)

    def body(q_ref, k_ref, v_ref, v8_ref, o_ref, ko_ref, vo_ref, obuf_ref,
             oin_ref, k_send, k_recv, v_send, v_recv, o_send, o_recv):
        x_idx = lax.axis_index("x")
        my_y = lax.axis_index("y")

        barrier = pltpu.get_barrier_semaphore()
        for nbr in ((x_idx, 1 - my_y), (1 - x_idx, my_y)):
            pl.semaphore_signal(
                barrier, inc=1, device_id=nbr,
                device_id_type=pl.DeviceIdType.MESH,
            )
        pl.semaphore_wait(barrier, 2)

        def run_column(x):
            y_peer = (x, 1 - my_y)
            x_peer = (1 - x, my_y)
            mine_off = x * SHARE
            twin_off = (1 - x) * SHARE

            def rdma(src, dst, send, recv, i, peer):
                return pltpu.make_async_remote_copy(
                    src_ref=src.at[i], dst_ref=dst.at[i],
                    send_sem=send.at[i], recv_sem=recv.at[i],
                    device_id=peer, device_id_type=pl.DeviceIdType.MESH,
                )

            kx = [
                rdma(k_ref, ko_ref, k_send, k_recv, i, y_peer)
                for i in range(SHARE)
            ]
            vx = [
                rdma(v8_ref, vo_ref, v_send, v_recv, i, y_peer)
                for i in range(SHARE)
            ]
            for i in range(SHARE):
                kx[i].start()
                vx[i].start()

            o_ship = []
            for i in range(SHARE):
                qh = q_ref[:, i, :]
                s0 = lax.dot_general(
                    qh, k_ref[i], (((1,), (1,)), ((), ())),
                    preferred_element_type=jnp.float32,
                )
                p0 = jnp.exp(s0)
                l0 = jnp.sum(p0, axis=1, keepdims=True)
                o0 = lax.dot_general(
                    p0.astype(jnp.bfloat16), v_ref[i],
                    (((1,), (0,)), ((), ())),
                    preferred_element_type=jnp.float32,
                )
                kx[i].wait()
                s1 = lax.dot_general(
                    qh, ko_ref[i], (((1,), (1,)), ((), ())),
                    preferred_element_type=jnp.float32,
                )
                p1 = jnp.exp(s1)
                l1 = jnp.sum(p1, axis=1, keepdims=True)
                vx[i].wait()
                o1 = lax.dot_general(
                    p1.astype(jnp.bfloat16), vo_ref[i].astype(jnp.bfloat16),
                    (((1,), (0,)), ((), ())),
                    preferred_element_type=jnp.float32,
                )
                o_head = (o0 + o1) / (l0 + l1)
                o_ref[:, mine_off + i, :] = o_head
                obuf_ref[i] = o_head.astype(jnp.bfloat16)
                ship = rdma(obuf_ref, oin_ref, o_send, o_recv, i, x_peer)
                ship.start()
                o_ship.append(ship)

            for i in range(SHARE):
                rdma(obuf_ref, oin_ref, o_send, o_recv, i, x_peer).wait_recv()
                o_ref[:, twin_off + i, :] = oin_ref[i].astype(jnp.float32)

            for ship in o_ship:
                ship.wait_send()

        @pl.when(x_idx == 0)
        def _():
            run_column(0)

        @pl.when(x_idx == 1)
        def _():
            run_column(1)

    o = pl.pallas_call(
        body,
        out_shape=jax.ShapeDtypeStruct((S_LOCAL, H, D), jnp.float32),
        in_specs=[pl.BlockSpec(memory_space=pltpu.VMEM)] * 4,
        out_specs=pl.BlockSpec(memory_space=pltpu.VMEM),
        scratch_shapes=[
            pltpu.VMEM((SHARE, S_LOCAL, D), jnp.bfloat16),
            pltpu.VMEM((SHARE, S_LOCAL, D), jnp.float8_e4m3fn),
            pltpu.VMEM((SHARE, S_LOCAL, D), jnp.bfloat16),
            pltpu.VMEM((SHARE, S_LOCAL, D), jnp.bfloat16),
            pltpu.SemaphoreType.DMA((SHARE,)),
            pltpu.SemaphoreType.DMA((SHARE,)),
            pltpu.SemaphoreType.DMA((SHARE,)),
            pltpu.SemaphoreType.DMA((SHARE,)),
            pltpu.SemaphoreType.DMA((SHARE,)),
            pltpu.SemaphoreType.DMA((SHARE,)),
        ],
        compiler_params=pltpu.CompilerParams(
            collective_id=0, vmem_limit_bytes=60 * 1024 * 1024
        ),
    )(q, k, v, v8)

    return o[None]
